# baseline (speedup 1.0000x reference)
import jax
import jax.numpy as jnp
from jax import lax
from jax.experimental import pallas as pl
from jax.experimental.pallas import tpu as pltpu
from jax.experimental.pallas import tpu_sc as plsc

L = 16
NC = 2
NS = 16
NW = NC * NS
CHUNK = 128
N_COMP = 5
C_DIM = 64
RING = 10


def _tce_body(x_hbm, comp_hbm, e0, e1, e2, e3, e4,
              o0, o1, o2, o3, o4,
              x_v, cidx_v, cvals_v, rows_v, semc, semg, semw):
    embs = (e0, e1, e2, e3, e4)
    outs = (o0, o1, o2, o3, o4)
    batch = x_hbm.shape[0]
    t_vocab = comp_hbm.shape[0] // N_COMP
    per_w = batch // NW
    nch = per_w // CHUNK

    wid = lax.axis_index("s") * NC + lax.axis_index("c")
    base = wid * per_w

    pltpu.sync_copy(x_hbm.at[pl.ds(base, per_w)], x_v)
    for i in range(N_COMP):
        off = jnp.int32(i * t_vocab)
        for j in range(per_w // L):
            cidx_v[pl.ds(i * per_w + j * L, L)] = x_v[pl.ds(j * L, L)] + off

    comp_dmas = []
    for i in range(N_COMP):
        for c in range(nch):
            o = i * per_w + c * CHUNK
            comp_dmas.append(pltpu.async_copy(
                comp_hbm.at[cidx_v.at[pl.ds(o, CHUNK)]],
                cvals_v.at[pl.ds(o, CHUNK)], semc))
    for d in comp_dmas:
        d.wait()

    tasks = [(i, c) for i in range(N_COMP) for c in range(nch)]
    n = len(tasks)
    gd = [None] * n
    wd = [None] * n

    def fire(t):
        i, c = tasks[t]
        o = i * per_w + c * CHUNK
        gd[t] = pltpu.async_copy(
            embs[i].at[cvals_v.at[pl.ds(o, CHUNK)]],
            rows_v.at[t % RING], semg.at[t % RING])

    for t in range(min(RING, n)):
        fire(t)
    for t in range(n):
        gd[t].wait()
        i, c = tasks[t]
        wd[t] = pltpu.async_copy(
            rows_v.at[t % RING], outs[i].at[pl.ds(base + c * CHUNK, CHUNK)],
            semw.at[t % RING])
        if t + RING < n:
            wd[t].wait()
            fire(t + RING)
    for t in range(max(0, n - RING), n):
        wd[t].wait()


def kernel(x, comp_table, emb0, emb1, emb2, emb3, emb4):
    batch = x.shape[0]
    per_w = batch // NW
    comp_cm = comp_table.T.reshape(-1)
    embs = tuple(e.at[0].set(0.0) for e in (emb0, emb1, emb2, emb3, emb4))

    mesh = plsc.VectorSubcoreMesh(core_axis_name="c", subcore_axis_name="s")
    out_type = tuple(
        jax.ShapeDtypeStruct((batch, C_DIM), jnp.float32) for _ in range(N_COMP)
    )
    scratch = [
        pltpu.VMEM((per_w,), jnp.int32),
        pltpu.VMEM((N_COMP * per_w,), jnp.int32),
        pltpu.VMEM((N_COMP * per_w,), jnp.int32),
        pltpu.VMEM((RING, CHUNK, C_DIM), jnp.float32),
        pltpu.SemaphoreType.DMA,
        pltpu.SemaphoreType.DMA((RING,)),
        pltpu.SemaphoreType.DMA((RING,)),
    ]
    f = pl.kernel(
        _tce_body, mesh=mesh, out_type=out_type, scratch_types=scratch,
        compiler_params=pltpu.CompilerParams(use_tc_tiling_on_sc=False),
    )
    return f(x, comp_cm, *embs)
